# Initial kernel scaffold; baseline (speedup 1.0000x reference)
import jax
import jax.numpy as jnp
from jax.experimental import pallas as pl


def kernel(x, edge_index, edge_attr, W1, b1, W2, b2, W3, b3, W4, b4):
    raise NotImplementedError("write your pallas kernel here")


# speedup vs baseline: 22.9907x; 22.9907x over previous
import functools

import jax
import jax.numpy as jnp
from jax import lax
from jax.experimental import pallas as pl
from jax.experimental.pallas import tpu as pltpu
from jax.experimental.pallas import tpu_sc as plsc

F = 16
SUB = 128
K = 16
NCORES = 2
NSUB = 16
NTILES = NCORES * NSUB
BT = 4000


def _agg_body(n_chunks, rpt,
              z_hbm, row_hbm, col_hbm, zeros_hbm, out,
              row_v, col_v, buf, acc, gsem):
    c = lax.axis_index("c")
    s = lax.axis_index("s")
    wid = s * NCORES + c
    pltpu.sync_copy(zeros_hbm, acc.at[pl.ds(s * rpt, rpt)])
    plsc.subcore_barrier()
    row_base = wid * (n_chunks * K)

    def chunk(g, carry):
        r0 = row_base + g * K
        pltpu.sync_copy(row_hbm.at[pl.ds(r0, K)], row_v)
        pltpu.sync_copy(col_hbm.at[pl.ds(r0, K)], col_v)

        def sub(j, carry2):
            pltpu.async_copy(z_hbm.at[row_v.at[j]], buf, gsem).wait()
            pltpu.sync_copy(buf, acc.at[col_v.at[j]], add=True)
            return carry2

        return lax.fori_loop(0, K, sub, carry)

    lax.fori_loop(0, n_chunks, chunk, 0)
    plsc.subcore_barrier()
    pltpu.sync_copy(acc.at[pl.ds(s * rpt, rpt)],
                    out.at[c, pl.ds(s * rpt, rpt)])


def _agg_call(z, row2, col2, zeros16, n_chunks, rpt):
    n_pad = rpt * NSUB
    mesh = plsc.VectorSubcoreMesh(core_axis_name="c", subcore_axis_name="s")
    return pl.kernel(
        functools.partial(_agg_body, n_chunks, rpt),
        out_type=jax.ShapeDtypeStruct((NCORES, n_pad, F), jnp.float32),
        mesh=mesh,
        compiler_params=pltpu.CompilerParams(use_tc_tiling_on_sc=False),
        scratch_types=[
            pltpu.VMEM((K, SUB), jnp.int32),
            pltpu.VMEM((K, SUB), jnp.int32),
            pltpu.VMEM((SUB, F), jnp.float32),
            pltpu.VMEM_SHARED((n_pad, F), jnp.float32),
            pltpu.SemaphoreType.DMA,
        ],
    )(z, row2, col2, zeros16)


def _dense1_body(d0, d1, x_ref, w_ref, dis_ref, z_ref):
    deg = d0[0][:, 0:1] + d1[0][:, 0:1] + 1.0
    dis = lax.rsqrt(deg)
    dis_ref[...] = dis
    z_ref[...] = jnp.dot(x_ref[...], w_ref[...],
                         preferred_element_type=jnp.float32) * dis


def _dense1(degp, x, w1):
    n, fin = x.shape
    grid = (n // BT,)
    return pl.pallas_call(
        _dense1_body,
        grid=grid,
        in_specs=[
            pl.BlockSpec((1, BT, F), lambda i: (0, i, 0)),
            pl.BlockSpec((1, BT, F), lambda i: (1, i, 0)),
            pl.BlockSpec((BT, fin), lambda i: (i, 0)),
            pl.BlockSpec((fin, F), lambda i: (0, 0)),
        ],
        out_specs=[
            pl.BlockSpec((BT, 1), lambda i: (i, 0)),
            pl.BlockSpec((BT, F), lambda i: (i, 0)),
        ],
        out_shape=[
            jax.ShapeDtypeStruct((n, 1), jnp.float32),
            jax.ShapeDtypeStruct((n, F), jnp.float32),
        ],
    )(degp, degp, x, w1)


def _mid_body(s0, s1, z, dis, b_ref, w_ref, zn_ref):
    h = jnp.maximum(dis[...] * (s0[0] + s1[0] + z[...]) + b_ref[...], 0.0)
    zn_ref[...] = jnp.dot(h, w_ref[...],
                          preferred_element_type=jnp.float32) * dis[...]


def _mid(sp, z, dis, b, w):
    n = z.shape[0]
    grid = (n // BT,)
    return pl.pallas_call(
        _mid_body,
        grid=grid,
        in_specs=[
            pl.BlockSpec((1, BT, F), lambda i: (0, i, 0)),
            pl.BlockSpec((1, BT, F), lambda i: (1, i, 0)),
            pl.BlockSpec((BT, F), lambda i: (i, 0)),
            pl.BlockSpec((BT, 1), lambda i: (i, 0)),
            pl.BlockSpec((1, F), lambda i: (0, 0)),
            pl.BlockSpec((F, F), lambda i: (0, 0)),
        ],
        out_specs=pl.BlockSpec((BT, F), lambda i: (i, 0)),
        out_shape=jax.ShapeDtypeStruct((n, F), jnp.float32),
    )(sp, sp, z, dis, b.reshape(1, F), w)


def _final_body(s0, s1, a4, dis, w4_ref, b4_ref, o_ref):
    g = dis[...] * (s0[0] + s1[0] + a4[...])
    h = jnp.dot(g, w4_ref[...], preferred_element_type=jnp.float32) + b4_ref[...]
    m = jnp.max(h, axis=1, keepdims=True)
    lse = m + jnp.log(jnp.sum(jnp.exp(h - m), axis=1, keepdims=True))
    o_ref[...] = h - lse


def _final(sp, a4, dis, w4, b4):
    n = a4.shape[0]
    fo = w4.shape[1]
    grid = (n // BT,)
    return pl.pallas_call(
        _final_body,
        grid=grid,
        in_specs=[
            pl.BlockSpec((1, BT, F), lambda i: (0, i, 0)),
            pl.BlockSpec((1, BT, F), lambda i: (1, i, 0)),
            pl.BlockSpec((BT, F), lambda i: (i, 0)),
            pl.BlockSpec((BT, 1), lambda i: (i, 0)),
            pl.BlockSpec((F, fo), lambda i: (0, 0)),
            pl.BlockSpec((1, fo), lambda i: (0, 0)),
        ],
        out_specs=pl.BlockSpec((BT, fo), lambda i: (i, 0)),
        out_shape=jax.ShapeDtypeStruct((n, fo), jnp.float32),
    )(sp, sp, a4, dis, w4, b4.reshape(1, fo))


def kernel(x, edge_index, edge_attr, W1, b1, W2, b2, W3, b3, W4, b4):
    n = x.shape[0]
    e = edge_index.shape[1]
    assert n % BT == 0 and n % NSUB == 0

    row = edge_index[0].astype(jnp.int32)
    col = edge_index[1].astype(jnp.int32)

    ch = NTILES * SUB * K
    n_chunks = -(-e // ch)
    e_pad = n_chunks * ch
    padn = e_pad - e
    if padn:
        row = jnp.concatenate([row, jnp.zeros((padn,), jnp.int32)])
        col = jnp.concatenate([col, jnp.full((padn,), n, jnp.int32)])
    row2 = row.reshape(-1, SUB)
    col2 = col.reshape(-1, SUB)

    rpt = (-(-(n + 1) // NSUB) + 7) // 8 * 8

    zeros16 = jnp.zeros((rpt, F), jnp.float32)
    ones16 = jnp.ones((n, F), jnp.float32)

    degp = _agg_call(ones16, row2, col2, zeros16, n_chunks, rpt)
    dis, z1 = _dense1(degp, x, W1)
    s1 = _agg_call(z1, row2, col2, zeros16, n_chunks, rpt)
    z2 = _mid(s1, z1, dis, b1, W2)
    s2 = _agg_call(z2, row2, col2, zeros16, n_chunks, rpt)
    z3 = _mid(s2, z2, dis, b2, W3)
    s3 = _agg_call(z3, row2, col2, zeros16, n_chunks, rpt)
    a4 = _mid(s3, z3, dis, b3, jnp.eye(F, dtype=jnp.float32))
    s4 = _agg_call(a4, row2, col2, zeros16, n_chunks, rpt)
    return _final(s4, a4, dis, W4, b4)


# speedup vs baseline: 36.5719x; 1.5907x over previous
import functools

import jax
import jax.numpy as jnp
from jax import lax
from jax.experimental import pallas as pl
from jax.experimental.pallas import tpu as pltpu
from jax.experimental.pallas import tpu_sc as plsc

F = 16
W8 = 8
SUB = 128
K = 4
NCORES = 2
NSUB = 16
NTILES = NCORES * NSUB
BT = 4000

_SC_PARAMS = pltpu.CompilerParams(use_tc_tiling_on_sc=False)
_MESH = dict(core_axis_name="c", subcore_axis_name="s")


def _agg_body(n_chunks, rpt, w,
              z_hbm, row_hbm, col_hbm, zeros_hbm, out,
              row_v, col_v, bufs, acc, gsem0, gsem1, ssem0, ssem1):
    c = lax.axis_index("c")
    s = lax.axis_index("s")
    wid = s * NCORES + c
    pltpu.sync_copy(zeros_hbm, acc.at[pl.ds(s * rpt, rpt)])
    plsc.subcore_barrier()
    row_base = wid * n_chunks
    dummy = z_hbm.at[pl.ds(0, SUB)]

    def drain(sem, dst, m):
        def d(i, carry):
            pltpu.make_async_copy(dummy, dst, sem).wait()
            return carry
        lax.fori_loop(0, m, d, 0)

    def run_chunk(g, p, gsem, ssem):
        pltpu.sync_copy(row_hbm.at[row_base + g], row_v.at[p])
        pltpu.sync_copy(col_hbm.at[row_base + g], col_v.at[p])

        def fire_gather(j, carry):
            pltpu.async_copy(z_hbm.at[row_v.at[p, j]], bufs.at[p, j], gsem)
            return carry
        lax.fori_loop(0, K, fire_gather, 0)
        drain(gsem, bufs.at[p, 0], K)

        def fire_scatter(j, carry):
            pltpu.async_copy(bufs.at[p, j], acc.at[col_v.at[p, j]], ssem,
                             add=True)
            return carry
        lax.fori_loop(0, K, fire_scatter, 0)

    def pair(gg, carry):
        @pl.when(gg > 0)
        def _():
            drain(ssem0, bufs.at[0, 0], K)
        run_chunk(2 * gg, 0, gsem0, ssem0)

        @pl.when(gg > 0)
        def _():
            drain(ssem1, bufs.at[1, 0], K)
        run_chunk(2 * gg + 1, 1, gsem1, ssem1)
        return carry

    lax.fori_loop(0, n_chunks // 2, pair, 0)
    drain(ssem0, bufs.at[0, 0], K)
    drain(ssem1, bufs.at[1, 0], K)
    plsc.subcore_barrier()
    pltpu.sync_copy(acc.at[pl.ds(s * rpt, rpt)],
                    out.at[c, pl.ds(s * rpt, rpt)])


def _agg_call(z, row2, col2, zeros, n_chunks, rpt):
    w = z.shape[1]
    n_pad = rpt * NSUB
    return pl.kernel(
        functools.partial(_agg_body, n_chunks, rpt, w),
        out_type=jax.ShapeDtypeStruct((NCORES, n_pad, w), jnp.float32),
        mesh=plsc.VectorSubcoreMesh(**_MESH),
        compiler_params=_SC_PARAMS,
        scratch_types=[
            pltpu.VMEM((2, K, SUB), jnp.int32),
            pltpu.VMEM((2, K, SUB), jnp.int32),
            pltpu.VMEM((2, K, SUB, w), jnp.float32),
            pltpu.VMEM_SHARED((n_pad, w), jnp.float32),
            pltpu.SemaphoreType.DMA,
            pltpu.SemaphoreType.DMA,
            pltpu.SemaphoreType.DMA,
            pltpu.SemaphoreType.DMA,
        ],
    )(z, row2, col2, zeros)


def _deg_body(n_chunks, rpt,
              col_hbm, zeros_hbm, ones_hbm, out,
              col_v, ones_v, acc, ssem):
    c = lax.axis_index("c")
    s = lax.axis_index("s")
    wid = s * NCORES + c
    pltpu.sync_copy(zeros_hbm, acc.at[pl.ds(s * rpt, rpt)])
    pltpu.sync_copy(ones_hbm, ones_v)
    plsc.subcore_barrier()
    row_base = wid * n_chunks

    def chunk(g, carry):
        pltpu.sync_copy(col_hbm.at[row_base + g], col_v)

        def fire(j, carry2):
            pltpu.async_copy(ones_v, acc.at[col_v.at[j]], ssem, add=True)
            return carry2
        return lax.fori_loop(0, K, fire, carry)

    lax.fori_loop(0, n_chunks, chunk, 0)

    def d(i, carry):
        pltpu.make_async_copy(ones_hbm, ones_v, ssem).wait()
        return carry
    lax.fori_loop(0, n_chunks * K, d, 0)
    plsc.subcore_barrier()
    pltpu.sync_copy(acc.at[pl.ds(s * rpt, rpt)],
                    out.at[c, pl.ds(s * rpt, rpt)])


def _deg_call(col2, zeros8, ones8, n_chunks, rpt):
    n_pad = rpt * NSUB
    return pl.kernel(
        functools.partial(_deg_body, n_chunks, rpt),
        out_type=jax.ShapeDtypeStruct((NCORES, n_pad, W8), jnp.float32),
        mesh=plsc.VectorSubcoreMesh(**_MESH),
        compiler_params=_SC_PARAMS,
        scratch_types=[
            pltpu.VMEM((K, SUB), jnp.int32),
            pltpu.VMEM((SUB, W8), jnp.float32),
            pltpu.VMEM_SHARED((n_pad, W8), jnp.float32),
            pltpu.SemaphoreType.DMA,
        ],
    )(col2, zeros8, ones8)


def _dense1_body(d0, d1, xp_ref, dis_ref, z_ref):
    deg = d0[0][:, 0:1] + d1[0][:, 0:1] + 1.0
    dis = lax.rsqrt(deg)
    dis_ref[...] = dis
    z_ref[...] = xp_ref[...] * dis


def _dense1(degp, xp):
    n = xp.shape[0]
    grid = (n // BT,)
    return pl.pallas_call(
        _dense1_body,
        grid=grid,
        in_specs=[
            pl.BlockSpec((1, BT, W8), lambda i: (0, i, 0)),
            pl.BlockSpec((1, BT, W8), lambda i: (1, i, 0)),
            pl.BlockSpec((BT, W8), lambda i: (i, 0)),
        ],
        out_specs=[
            pl.BlockSpec((BT, 1), lambda i: (i, 0)),
            pl.BlockSpec((BT, W8), lambda i: (i, 0)),
        ],
        out_shape=[
            jax.ShapeDtypeStruct((n, 1), jnp.float32),
            jax.ShapeDtypeStruct((n, W8), jnp.float32),
        ],
    )(degp, degp, xp)


def _dense2_body(s0, s1, z, dis, w1_ref, b1_ref, w2_ref, zn_ref):
    g = dis[...] * (s0[0] + s1[0] + z[...])
    h1 = jnp.maximum(jnp.dot(g, w1_ref[...],
                             preferred_element_type=jnp.float32) + b1_ref[...],
                     0.0)
    zn_ref[...] = jnp.dot(h1, w2_ref[...],
                          preferred_element_type=jnp.float32) * dis[...]


def _dense2(sp, z, dis, w1p, b1, w2):
    n = z.shape[0]
    grid = (n // BT,)
    return pl.pallas_call(
        _dense2_body,
        grid=grid,
        in_specs=[
            pl.BlockSpec((1, BT, W8), lambda i: (0, i, 0)),
            pl.BlockSpec((1, BT, W8), lambda i: (1, i, 0)),
            pl.BlockSpec((BT, W8), lambda i: (i, 0)),
            pl.BlockSpec((BT, 1), lambda i: (i, 0)),
            pl.BlockSpec((W8, F), lambda i: (0, 0)),
            pl.BlockSpec((1, F), lambda i: (0, 0)),
            pl.BlockSpec((F, F), lambda i: (0, 0)),
        ],
        out_specs=pl.BlockSpec((BT, F), lambda i: (i, 0)),
        out_shape=jax.ShapeDtypeStruct((n, F), jnp.float32),
    )(sp, sp, z, dis, w1p, b1.reshape(1, F), w2)


def _mid_body(s0, s1, z, dis, b_ref, w_ref, zn_ref):
    h = jnp.maximum(dis[...] * (s0[0] + s1[0] + z[...]) + b_ref[...], 0.0)
    zn_ref[...] = jnp.dot(h, w_ref[...],
                          preferred_element_type=jnp.float32) * dis[...]


def _mid(sp, z, dis, b, w):
    n = z.shape[0]
    wo = w.shape[1]
    grid = (n // BT,)
    return pl.pallas_call(
        _mid_body,
        grid=grid,
        in_specs=[
            pl.BlockSpec((1, BT, F), lambda i: (0, i, 0)),
            pl.BlockSpec((1, BT, F), lambda i: (1, i, 0)),
            pl.BlockSpec((BT, F), lambda i: (i, 0)),
            pl.BlockSpec((BT, 1), lambda i: (i, 0)),
            pl.BlockSpec((1, F), lambda i: (0, 0)),
            pl.BlockSpec((F, wo), lambda i: (0, 0)),
        ],
        out_specs=pl.BlockSpec((BT, wo), lambda i: (i, 0)),
        out_shape=jax.ShapeDtypeStruct((n, wo), jnp.float32),
    )(sp, sp, z, dis, b.reshape(1, F), w)


def _final_body(s0, s1, z, dis, b4_ref, o_ref):
    g = dis[...] * (s0[0] + s1[0] + z[...])
    h = g[:, 0:2] + b4_ref[...]
    m = jnp.max(h, axis=1, keepdims=True)
    lse = m + jnp.log(jnp.sum(jnp.exp(h - m), axis=1, keepdims=True))
    o_ref[...] = h - lse


def _final(sp, z, dis, b4):
    n = z.shape[0]
    fo = b4.shape[0]
    grid = (n // BT,)
    return pl.pallas_call(
        _final_body,
        grid=grid,
        in_specs=[
            pl.BlockSpec((1, BT, W8), lambda i: (0, i, 0)),
            pl.BlockSpec((1, BT, W8), lambda i: (1, i, 0)),
            pl.BlockSpec((BT, W8), lambda i: (i, 0)),
            pl.BlockSpec((BT, 1), lambda i: (i, 0)),
            pl.BlockSpec((1, fo), lambda i: (0, 0)),
        ],
        out_specs=pl.BlockSpec((BT, fo), lambda i: (i, 0)),
        out_shape=jax.ShapeDtypeStruct((n, fo), jnp.float32),
    )(sp, sp, z, dis, b4.reshape(1, fo))


def kernel(x, edge_index, edge_attr, W1, b1, W2, b2, W3, b3, W4, b4):
    n = x.shape[0]
    e = edge_index.shape[1]
    assert n % BT == 0 and n % NSUB == 0

    row = edge_index[0].astype(jnp.int32)
    col = edge_index[1].astype(jnp.int32)

    ch = NTILES * SUB * K
    n_chunks = -(-e // ch)
    n_chunks += n_chunks % 2
    e_pad = n_chunks * ch
    padn = e_pad - e
    rpt = (-(-(n + 1) // NSUB) + 7) // 8 * 8
    n_pad = rpt * NSUB
    if padn:
        row = jnp.concatenate([row, jnp.zeros((padn,), jnp.int32)])
        spread = n + jnp.arange(padn, dtype=jnp.int32) % (n_pad - n)
        col = jnp.concatenate([col, spread])
    row2 = row.reshape(-1, K, SUB)
    col2 = col.reshape(-1, K, SUB)

    zeros16 = jnp.zeros((rpt, F), jnp.float32)
    zeros8 = jnp.zeros((rpt, W8), jnp.float32)
    ones8 = jnp.ones((SUB, W8), jnp.float32)
    xp = jnp.pad(x, ((0, 0), (0, W8 - x.shape[1])))
    w1p = jnp.pad(W1, ((0, W8 - W1.shape[0]), (0, 0)))
    w4p = jnp.pad(W4, ((0, 0), (0, W8 - W4.shape[1])))

    degp = _deg_call(col2, zeros8, ones8, n_chunks, rpt)
    dis, z1 = _dense1(degp, xp)
    s1 = _agg_call(z1, row2, col2, zeros8, n_chunks, rpt)
    z2 = _dense2(s1, z1, dis, w1p, b1, W2)
    s2 = _agg_call(z2, row2, col2, zeros16, n_chunks, rpt)
    z3 = _mid(s2, z2, dis, b2, W3)
    s3 = _agg_call(z3, row2, col2, zeros16, n_chunks, rpt)
    z4 = _mid(s3, z3, dis, b3, w4p)
    s4 = _agg_call(z4, row2, col2, zeros8, n_chunks, rpt)
    return _final(s4, z4, dis, b4)


# speedup vs baseline: 36.7109x; 1.0038x over previous
import functools

import jax
import jax.numpy as jnp
from jax import lax
from jax.experimental import pallas as pl
from jax.experimental.pallas import tpu as pltpu
from jax.experimental.pallas import tpu_sc as plsc

F = 16
W8 = 8
SUB = 512
K = 1
NCORES = 2
NSUB = 16
NTILES = NCORES * NSUB
BT = 4000

_SC_PARAMS = pltpu.CompilerParams(use_tc_tiling_on_sc=False)
_MESH = dict(core_axis_name="c", subcore_axis_name="s")


def _agg_body(n_chunks, rpt, w,
              z_hbm, row_hbm, col_hbm, zeros_hbm, out,
              row_v, col_v, bufs, acc, gsem0, gsem1, ssem0, ssem1):
    c = lax.axis_index("c")
    s = lax.axis_index("s")
    wid = s * NCORES + c
    pltpu.sync_copy(zeros_hbm, acc.at[pl.ds(s * rpt, rpt)])
    plsc.subcore_barrier()
    row_base = wid * n_chunks
    dummy = z_hbm.at[pl.ds(0, SUB)]

    def drain(sem, dst, m):
        def d(i, carry):
            pltpu.make_async_copy(dummy, dst, sem).wait()
            return carry
        lax.fori_loop(0, m, d, 0)

    def run_chunk(g, p, gsem, ssem):
        pltpu.sync_copy(row_hbm.at[row_base + g], row_v.at[p])
        pltpu.sync_copy(col_hbm.at[row_base + g], col_v.at[p])

        def fire_gather(j, carry):
            pltpu.async_copy(z_hbm.at[row_v.at[p, j]], bufs.at[p, j], gsem)
            return carry
        lax.fori_loop(0, K, fire_gather, 0)
        drain(gsem, bufs.at[p, 0], K)

        def fire_scatter(j, carry):
            pltpu.async_copy(bufs.at[p, j], acc.at[col_v.at[p, j]], ssem,
                             add=True)
            return carry
        lax.fori_loop(0, K, fire_scatter, 0)

    def pair(gg, carry):
        @pl.when(gg > 0)
        def _():
            drain(ssem0, bufs.at[0, 0], K)
        run_chunk(2 * gg, 0, gsem0, ssem0)

        @pl.when(gg > 0)
        def _():
            drain(ssem1, bufs.at[1, 0], K)
        run_chunk(2 * gg + 1, 1, gsem1, ssem1)
        return carry

    lax.fori_loop(0, n_chunks // 2, pair, 0)
    drain(ssem0, bufs.at[0, 0], K)
    drain(ssem1, bufs.at[1, 0], K)
    plsc.subcore_barrier()
    pltpu.sync_copy(acc.at[pl.ds(s * rpt, rpt)],
                    out.at[c, pl.ds(s * rpt, rpt)])


def _agg_call(z, row2, col2, zeros, n_chunks, rpt):
    w = z.shape[1]
    n_pad = rpt * NSUB
    return pl.kernel(
        functools.partial(_agg_body, n_chunks, rpt, w),
        out_type=jax.ShapeDtypeStruct((NCORES, n_pad, w), jnp.float32),
        mesh=plsc.VectorSubcoreMesh(**_MESH),
        compiler_params=_SC_PARAMS,
        scratch_types=[
            pltpu.VMEM((2, K, SUB), jnp.int32),
            pltpu.VMEM((2, K, SUB), jnp.int32),
            pltpu.VMEM((2, K, SUB, w), jnp.float32),
            pltpu.VMEM_SHARED((n_pad, w), jnp.float32),
            pltpu.SemaphoreType.DMA,
            pltpu.SemaphoreType.DMA,
            pltpu.SemaphoreType.DMA,
            pltpu.SemaphoreType.DMA,
        ],
    )(z, row2, col2, zeros)


def _deg_body(n_chunks, rpt,
              col_hbm, zeros_hbm, ones_hbm, out,
              col_v, ones_v, acc, ssem):
    c = lax.axis_index("c")
    s = lax.axis_index("s")
    wid = s * NCORES + c
    pltpu.sync_copy(zeros_hbm, acc.at[pl.ds(s * rpt, rpt)])
    pltpu.sync_copy(ones_hbm, ones_v)
    plsc.subcore_barrier()
    row_base = wid * n_chunks

    def chunk(g, carry):
        pltpu.sync_copy(col_hbm.at[row_base + g], col_v)

        def fire(j, carry2):
            pltpu.async_copy(ones_v, acc.at[col_v.at[j]], ssem, add=True)
            return carry2
        return lax.fori_loop(0, K, fire, carry)

    lax.fori_loop(0, n_chunks, chunk, 0)

    def d(i, carry):
        pltpu.make_async_copy(ones_hbm, ones_v, ssem).wait()
        return carry
    lax.fori_loop(0, n_chunks * K, d, 0)
    plsc.subcore_barrier()
    pltpu.sync_copy(acc.at[pl.ds(s * rpt, rpt)],
                    out.at[c, pl.ds(s * rpt, rpt)])


def _deg_call(col2, zeros8, ones8, n_chunks, rpt):
    n_pad = rpt * NSUB
    return pl.kernel(
        functools.partial(_deg_body, n_chunks, rpt),
        out_type=jax.ShapeDtypeStruct((NCORES, n_pad, W8), jnp.float32),
        mesh=plsc.VectorSubcoreMesh(**_MESH),
        compiler_params=_SC_PARAMS,
        scratch_types=[
            pltpu.VMEM((K, SUB), jnp.int32),
            pltpu.VMEM((SUB, W8), jnp.float32),
            pltpu.VMEM_SHARED((n_pad, W8), jnp.float32),
            pltpu.SemaphoreType.DMA,
        ],
    )(col2, zeros8, ones8)


def _dense1_body(d0, d1, xp_ref, dis_ref, z_ref):
    deg = d0[0][:, 0:1] + d1[0][:, 0:1] + 1.0
    dis = lax.rsqrt(deg)
    dis_ref[...] = dis
    z_ref[...] = xp_ref[...] * dis


def _dense1(degp, xp):
    n = xp.shape[0]
    grid = (n // BT,)
    return pl.pallas_call(
        _dense1_body,
        grid=grid,
        in_specs=[
            pl.BlockSpec((1, BT, W8), lambda i: (0, i, 0)),
            pl.BlockSpec((1, BT, W8), lambda i: (1, i, 0)),
            pl.BlockSpec((BT, W8), lambda i: (i, 0)),
        ],
        out_specs=[
            pl.BlockSpec((BT, 1), lambda i: (i, 0)),
            pl.BlockSpec((BT, W8), lambda i: (i, 0)),
        ],
        out_shape=[
            jax.ShapeDtypeStruct((n, 1), jnp.float32),
            jax.ShapeDtypeStruct((n, W8), jnp.float32),
        ],
    )(degp, degp, xp)


def _dense2_body(s0, s1, z, dis, w1_ref, b1_ref, w2_ref, zn_ref):
    g = dis[...] * (s0[0] + s1[0] + z[...])
    h1 = jnp.maximum(jnp.dot(g, w1_ref[...],
                             preferred_element_type=jnp.float32) + b1_ref[...],
                     0.0)
    zn_ref[...] = jnp.dot(h1, w2_ref[...],
                          preferred_element_type=jnp.float32) * dis[...]


def _dense2(sp, z, dis, w1p, b1, w2):
    n = z.shape[0]
    grid = (n // BT,)
    return pl.pallas_call(
        _dense2_body,
        grid=grid,
        in_specs=[
            pl.BlockSpec((1, BT, W8), lambda i: (0, i, 0)),
            pl.BlockSpec((1, BT, W8), lambda i: (1, i, 0)),
            pl.BlockSpec((BT, W8), lambda i: (i, 0)),
            pl.BlockSpec((BT, 1), lambda i: (i, 0)),
            pl.BlockSpec((W8, F), lambda i: (0, 0)),
            pl.BlockSpec((1, F), lambda i: (0, 0)),
            pl.BlockSpec((F, F), lambda i: (0, 0)),
        ],
        out_specs=pl.BlockSpec((BT, F), lambda i: (i, 0)),
        out_shape=jax.ShapeDtypeStruct((n, F), jnp.float32),
    )(sp, sp, z, dis, w1p, b1.reshape(1, F), w2)


def _mid_body(s0, s1, z, dis, b_ref, w_ref, zn_ref):
    h = jnp.maximum(dis[...] * (s0[0] + s1[0] + z[...]) + b_ref[...], 0.0)
    zn_ref[...] = jnp.dot(h, w_ref[...],
                          preferred_element_type=jnp.float32) * dis[...]


def _mid(sp, z, dis, b, w):
    n = z.shape[0]
    wo = w.shape[1]
    grid = (n // BT,)
    return pl.pallas_call(
        _mid_body,
        grid=grid,
        in_specs=[
            pl.BlockSpec((1, BT, F), lambda i: (0, i, 0)),
            pl.BlockSpec((1, BT, F), lambda i: (1, i, 0)),
            pl.BlockSpec((BT, F), lambda i: (i, 0)),
            pl.BlockSpec((BT, 1), lambda i: (i, 0)),
            pl.BlockSpec((1, F), lambda i: (0, 0)),
            pl.BlockSpec((F, wo), lambda i: (0, 0)),
        ],
        out_specs=pl.BlockSpec((BT, wo), lambda i: (i, 0)),
        out_shape=jax.ShapeDtypeStruct((n, wo), jnp.float32),
    )(sp, sp, z, dis, b.reshape(1, F), w)


def _final_body(s0, s1, z, dis, b4_ref, o_ref):
    g = dis[...] * (s0[0] + s1[0] + z[...])
    h = g[:, 0:2] + b4_ref[...]
    m = jnp.max(h, axis=1, keepdims=True)
    lse = m + jnp.log(jnp.sum(jnp.exp(h - m), axis=1, keepdims=True))
    o_ref[...] = h - lse


def _final(sp, z, dis, b4):
    n = z.shape[0]
    fo = b4.shape[0]
    grid = (n // BT,)
    return pl.pallas_call(
        _final_body,
        grid=grid,
        in_specs=[
            pl.BlockSpec((1, BT, W8), lambda i: (0, i, 0)),
            pl.BlockSpec((1, BT, W8), lambda i: (1, i, 0)),
            pl.BlockSpec((BT, W8), lambda i: (i, 0)),
            pl.BlockSpec((BT, 1), lambda i: (i, 0)),
            pl.BlockSpec((1, fo), lambda i: (0, 0)),
        ],
        out_specs=pl.BlockSpec((BT, fo), lambda i: (i, 0)),
        out_shape=jax.ShapeDtypeStruct((n, fo), jnp.float32),
    )(sp, sp, z, dis, b4.reshape(1, fo))


def kernel(x, edge_index, edge_attr, W1, b1, W2, b2, W3, b3, W4, b4):
    n = x.shape[0]
    e = edge_index.shape[1]
    assert n % BT == 0 and n % NSUB == 0

    row = edge_index[0].astype(jnp.int32)
    col = edge_index[1].astype(jnp.int32)

    ch = NTILES * SUB * K
    n_chunks = -(-e // ch)
    n_chunks += n_chunks % 2
    e_pad = n_chunks * ch
    padn = e_pad - e
    rpt = (-(-(n + 1) // NSUB) + 7) // 8 * 8
    n_pad = rpt * NSUB
    if padn:
        row = jnp.concatenate([row, jnp.zeros((padn,), jnp.int32)])
        spread = n + jnp.arange(padn, dtype=jnp.int32) % (n_pad - n)
        col = jnp.concatenate([col, spread])
    row2 = row.reshape(-1, K, SUB)
    col2 = col.reshape(-1, K, SUB)

    zeros16 = jnp.zeros((rpt, F), jnp.float32)
    zeros8 = jnp.zeros((rpt, W8), jnp.float32)
    ones8 = jnp.ones((SUB, W8), jnp.float32)
    xp = jnp.pad(x, ((0, 0), (0, W8 - x.shape[1])))
    w1p = jnp.pad(W1, ((0, W8 - W1.shape[0]), (0, 0)))
    w4p = jnp.pad(W4, ((0, 0), (0, W8 - W4.shape[1])))

    degp = _deg_call(col2, zeros8, ones8, n_chunks, rpt)
    dis, z1 = _dense1(degp, xp)
    s1 = _agg_call(z1, row2, col2, zeros8, n_chunks, rpt)
    z2 = _dense2(s1, z1, dis, w1p, b1, W2)
    s2 = _agg_call(z2, row2, col2, zeros16, n_chunks, rpt)
    z3 = _mid(s2, z2, dis, b2, W3)
    s3 = _agg_call(z3, row2, col2, zeros16, n_chunks, rpt)
    z4 = _mid(s3, z3, dis, b3, w4p)
    s4 = _agg_call(z4, row2, col2, zeros8, n_chunks, rpt)
    return _final(s4, z4, dis, b4)


# speedup vs baseline: 55.7723x; 1.5192x over previous
import functools

import jax
import jax.numpy as jnp
from jax import lax
from jax.experimental import pallas as pl
from jax.experimental.pallas import tpu as pltpu
from jax.experimental.pallas import tpu_sc as plsc

F = 16
W8 = 8
SUB = 512
K = 1
NCORES = 2
NSUB = 16
NTILES = NCORES * NSUB
BT = 4000

_SC_PARAMS = pltpu.CompilerParams(use_tc_tiling_on_sc=False)
_MESH = dict(core_axis_name="c", subcore_axis_name="s")


def _agg_body(n_chunks, rpt, w,
              z_hbm, row_hbm, col_hbm, zeros_hbm, out,
              row_v, col_v, bufs, acc, isem, gsem, ssem):
    c = lax.axis_index("c")
    s = lax.axis_index("s")
    wid = s * NCORES + c
    pltpu.sync_copy(zeros_hbm, acc.at[pl.ds(s * rpt, rpt)])
    plsc.subcore_barrier()
    base = wid * n_chunks
    dummy = z_hbm.at[pl.ds(0, SUB)]
    idummy = row_hbm.at[0]

    def fire_idx(g):
        gc = jnp.minimum(g, n_chunks - 1)
        m = lax.rem(gc, 4)
        pltpu.async_copy(row_hbm.at[base + gc], row_v.at[m], isem)
        pltpu.async_copy(col_hbm.at[base + gc], col_v.at[m], isem)

    fire_idx(0)

    def it(g, carry):
        p = lax.rem(g, 2)
        q = 1 - p

        @pl.when(jnp.logical_and(g >= 2, g < n_chunks))
        def _():
            pltpu.make_async_copy(dummy, bufs.at[p], ssem.at[p]).wait()

        @pl.when(g < n_chunks)
        def _():
            pltpu.make_async_copy(idummy, row_v.at[0], isem).wait()
            pltpu.make_async_copy(idummy, col_v.at[0], isem).wait()
            pltpu.async_copy(z_hbm.at[row_v.at[lax.rem(g, 4)]],
                             bufs.at[p], gsem.at[p])
            fire_idx(g + 1)

        @pl.when(g >= 1)
        def _():
            pltpu.make_async_copy(dummy, bufs.at[q], gsem.at[q]).wait()
            pltpu.async_copy(bufs.at[q], acc.at[col_v.at[lax.rem(g - 1, 4)]],
                             ssem.at[q], add=True)
        return carry

    lax.fori_loop(0, n_chunks + 1, it, 0)
    pltpu.make_async_copy(idummy, row_v.at[0], isem).wait()
    pltpu.make_async_copy(idummy, col_v.at[0], isem).wait()
    pltpu.make_async_copy(dummy, bufs.at[0], ssem.at[0]).wait()
    pltpu.make_async_copy(dummy, bufs.at[1], ssem.at[1]).wait()
    plsc.subcore_barrier()
    pltpu.sync_copy(acc.at[pl.ds(s * rpt, rpt)],
                    out.at[c, pl.ds(s * rpt, rpt)])


def _agg_call(z, row2, col2, zeros, n_chunks, rpt):
    w = z.shape[1]
    n_pad = rpt * NSUB
    return pl.kernel(
        functools.partial(_agg_body, n_chunks, rpt, w),
        out_type=jax.ShapeDtypeStruct((NCORES, n_pad, w), jnp.float32),
        mesh=plsc.VectorSubcoreMesh(**_MESH),
        compiler_params=_SC_PARAMS,
        scratch_types=[
            pltpu.VMEM((4, SUB), jnp.int32),
            pltpu.VMEM((4, SUB), jnp.int32),
            pltpu.VMEM((2, SUB, w), jnp.float32),
            pltpu.VMEM_SHARED((n_pad, w), jnp.float32),
            pltpu.SemaphoreType.DMA,
            pltpu.SemaphoreType.DMA((2,)),
            pltpu.SemaphoreType.DMA((2,)),
        ],
    )(z, row2, col2, zeros)


def _deg_body(n_chunks, rpt,
              col_hbm, zeros_hbm, ones_hbm, out,
              col_v, ones_v, acc, ssem):
    c = lax.axis_index("c")
    s = lax.axis_index("s")
    wid = s * NCORES + c
    pltpu.sync_copy(zeros_hbm, acc.at[pl.ds(s * rpt, rpt)])
    pltpu.sync_copy(ones_hbm, ones_v)
    plsc.subcore_barrier()
    row_base = wid * n_chunks

    def chunk(g, carry):
        pltpu.sync_copy(col_hbm.at[row_base + g], col_v)
        pltpu.async_copy(ones_v, acc.at[col_v], ssem, add=True)
        return carry

    lax.fori_loop(0, n_chunks, chunk, 0)

    def d(i, carry):
        pltpu.make_async_copy(ones_hbm, ones_v, ssem).wait()
        return carry
    lax.fori_loop(0, n_chunks, d, 0)
    plsc.subcore_barrier()
    pltpu.sync_copy(acc.at[pl.ds(s * rpt, rpt)],
                    out.at[c, pl.ds(s * rpt, rpt)])


def _deg_call(col2, zeros8, ones8, n_chunks, rpt):
    n_pad = rpt * NSUB
    return pl.kernel(
        functools.partial(_deg_body, n_chunks, rpt),
        out_type=jax.ShapeDtypeStruct((NCORES, n_pad, W8), jnp.float32),
        mesh=plsc.VectorSubcoreMesh(**_MESH),
        compiler_params=_SC_PARAMS,
        scratch_types=[
            pltpu.VMEM((SUB,), jnp.int32),
            pltpu.VMEM((SUB, W8), jnp.float32),
            pltpu.VMEM_SHARED((n_pad, W8), jnp.float32),
            pltpu.SemaphoreType.DMA,
        ],
    )(col2, zeros8, ones8)


def _dense1_body(d0, d1, xp_ref, dis_ref, z_ref):
    deg = d0[0][:, 0:1] + d1[0][:, 0:1] + 1.0
    dis = lax.rsqrt(deg)
    dis_ref[...] = dis
    z_ref[...] = xp_ref[...] * dis


def _dense1(degp, xp):
    n = xp.shape[0]
    grid = (n // BT,)
    return pl.pallas_call(
        _dense1_body,
        grid=grid,
        in_specs=[
            pl.BlockSpec((1, BT, W8), lambda i: (0, i, 0)),
            pl.BlockSpec((1, BT, W8), lambda i: (1, i, 0)),
            pl.BlockSpec((BT, W8), lambda i: (i, 0)),
        ],
        out_specs=[
            pl.BlockSpec((BT, 1), lambda i: (i, 0)),
            pl.BlockSpec((BT, W8), lambda i: (i, 0)),
        ],
        out_shape=[
            jax.ShapeDtypeStruct((n, 1), jnp.float32),
            jax.ShapeDtypeStruct((n, W8), jnp.float32),
        ],
    )(degp, degp, xp)


def _dense2_body(s0, s1, z, dis, w1_ref, b1_ref, w2_ref, zn_ref):
    g = dis[...] * (s0[0] + s1[0] + z[...])
    h1 = jnp.maximum(jnp.dot(g, w1_ref[...],
                             preferred_element_type=jnp.float32) + b1_ref[...],
                     0.0)
    zn_ref[...] = jnp.dot(h1, w2_ref[...],
                          preferred_element_type=jnp.float32) * dis[...]


def _dense2(sp, z, dis, w1p, b1, w2):
    n = z.shape[0]
    grid = (n // BT,)
    return pl.pallas_call(
        _dense2_body,
        grid=grid,
        in_specs=[
            pl.BlockSpec((1, BT, W8), lambda i: (0, i, 0)),
            pl.BlockSpec((1, BT, W8), lambda i: (1, i, 0)),
            pl.BlockSpec((BT, W8), lambda i: (i, 0)),
            pl.BlockSpec((BT, 1), lambda i: (i, 0)),
            pl.BlockSpec((W8, F), lambda i: (0, 0)),
            pl.BlockSpec((1, F), lambda i: (0, 0)),
            pl.BlockSpec((F, F), lambda i: (0, 0)),
        ],
        out_specs=pl.BlockSpec((BT, F), lambda i: (i, 0)),
        out_shape=jax.ShapeDtypeStruct((n, F), jnp.float32),
    )(sp, sp, z, dis, w1p, b1.reshape(1, F), w2)


def _mid_body(s0, s1, z, dis, b_ref, w_ref, zn_ref):
    h = jnp.maximum(dis[...] * (s0[0] + s1[0] + z[...]) + b_ref[...], 0.0)
    zn_ref[...] = jnp.dot(h, w_ref[...],
                          preferred_element_type=jnp.float32) * dis[...]


def _mid(sp, z, dis, b, w):
    n = z.shape[0]
    wo = w.shape[1]
    grid = (n // BT,)
    return pl.pallas_call(
        _mid_body,
        grid=grid,
        in_specs=[
            pl.BlockSpec((1, BT, F), lambda i: (0, i, 0)),
            pl.BlockSpec((1, BT, F), lambda i: (1, i, 0)),
            pl.BlockSpec((BT, F), lambda i: (i, 0)),
            pl.BlockSpec((BT, 1), lambda i: (i, 0)),
            pl.BlockSpec((1, F), lambda i: (0, 0)),
            pl.BlockSpec((F, wo), lambda i: (0, 0)),
        ],
        out_specs=pl.BlockSpec((BT, wo), lambda i: (i, 0)),
        out_shape=jax.ShapeDtypeStruct((n, wo), jnp.float32),
    )(sp, sp, z, dis, b.reshape(1, F), w)


def _final_body(s0, s1, z, dis, b4_ref, o_ref):
    g = dis[...] * (s0[0] + s1[0] + z[...])
    h = g[:, 0:2] + b4_ref[...]
    m = jnp.max(h, axis=1, keepdims=True)
    lse = m + jnp.log(jnp.sum(jnp.exp(h - m), axis=1, keepdims=True))
    o_ref[...] = h - lse


def _final(sp, z, dis, b4):
    n = z.shape[0]
    fo = b4.shape[0]
    grid = (n // BT,)
    return pl.pallas_call(
        _final_body,
        grid=grid,
        in_specs=[
            pl.BlockSpec((1, BT, W8), lambda i: (0, i, 0)),
            pl.BlockSpec((1, BT, W8), lambda i: (1, i, 0)),
            pl.BlockSpec((BT, W8), lambda i: (i, 0)),
            pl.BlockSpec((BT, 1), lambda i: (i, 0)),
            pl.BlockSpec((1, fo), lambda i: (0, 0)),
        ],
        out_specs=pl.BlockSpec((BT, fo), lambda i: (i, 0)),
        out_shape=jax.ShapeDtypeStruct((n, fo), jnp.float32),
    )(sp, sp, z, dis, b4.reshape(1, fo))


def kernel(x, edge_index, edge_attr, W1, b1, W2, b2, W3, b3, W4, b4):
    n = x.shape[0]
    e = edge_index.shape[1]
    assert n % BT == 0 and n % NSUB == 0

    row = edge_index[0].astype(jnp.int32)
    col = edge_index[1].astype(jnp.int32)

    ch = NTILES * SUB * K
    n_chunks = -(-e // ch)
    n_chunks += n_chunks % 2
    e_pad = n_chunks * ch
    padn = e_pad - e
    rpt = (-(-(n + 1) // NSUB) + 7) // 8 * 8
    n_pad = rpt * NSUB
    if padn:
        row = jnp.concatenate([row, jnp.zeros((padn,), jnp.int32)])
        spread = n + jnp.arange(padn, dtype=jnp.int32) % (n_pad - n)
        col = jnp.concatenate([col, spread])
    row2 = row.reshape(-1, SUB)
    col2 = col.reshape(-1, SUB)

    zeros16 = jnp.zeros((rpt, F), jnp.float32)
    zeros8 = jnp.zeros((rpt, W8), jnp.float32)
    ones8 = jnp.ones((SUB, W8), jnp.float32)
    xp = jnp.pad(x, ((0, 0), (0, W8 - x.shape[1])))
    w1p = jnp.pad(W1, ((0, W8 - W1.shape[0]), (0, 0)))
    w4p = jnp.pad(W4, ((0, 0), (0, W8 - W4.shape[1])))

    degp = _deg_call(col2, zeros8, ones8, n_chunks, rpt)
    dis, z1 = _dense1(degp, xp)
    s1 = _agg_call(z1, row2, col2, zeros8, n_chunks, rpt)
    z2 = _dense2(s1, z1, dis, w1p, b1, W2)
    s2 = _agg_call(z2, row2, col2, zeros16, n_chunks, rpt)
    z3 = _mid(s2, z2, dis, b2, W3)
    s3 = _agg_call(z3, row2, col2, zeros16, n_chunks, rpt)
    z4 = _mid(s3, z3, dis, b3, w4p)
    s4 = _agg_call(z4, row2, col2, zeros8, n_chunks, rpt)
    return _final(s4, z4, dis, b4)
